# baseline (speedup 1.0000x reference)
import functools

import jax
import jax.numpy as jnp
from jax import lax
from jax.experimental import pallas as pl
from jax.experimental.pallas import tpu as pltpu
from jax.experimental.pallas import tpu_sc as plsc

_N = 10000
_E = 320000
_D = 128
_H = 64

_NC = 2
_NS = 16
_NW = _NC * _NS
_EPT = _E // _NW
_CH = 80
_NCHUNK = _EPT // _CH
_NPAD = 10240
_RPT = _NPAD // _NS

_PERM = tuple(
    (c // 32) * 32 + (c % 32) // 2 + (16 if c % 2 else 0) for c in range(_H))

_GATHER_DNUMS = lax.GatherDimensionNumbers(
    offset_dims=(), collapsed_slice_dims=(0,), start_index_map=(0,))


def _splat_lane(vec, lane):
    idx = jnp.full((16, 1), lane, jnp.int32)
    return lax.gather(vec, idx, _GATHER_DNUMS, (1,),
                      mode=lax.GatherScatterMode.PROMISE_IN_BOUNDS)


_NBUF = 5
_NOUT = _NCHUNK // _NBUF


@functools.cache
def _make_sc_segment():
    mesh = plsc.VectorSubcoreMesh(core_axis_name="c", subcore_axis_name="s")
    return pl.kernel(
        _sc_segment_body,
        out_type=jax.ShapeDtypeStruct((_NC, _NS, _RPT, _H), jnp.float32),
        mesh=mesh,
        scratch_types=[
            pltpu.VMEM((_NCHUNK, _CH), jnp.int32),
            pltpu.VMEM((_NCHUNK, _CH), jnp.int32),
            pltpu.VMEM((_NCHUNK, _CH), jnp.float32),
            pltpu.VMEM((_NBUF, _CH, _H // 2), jnp.int32),
            pltpu.VMEM((_NBUF, _CH, _H), jnp.float32),
            pltpu.VMEM_SHARED((_NPAD, _H), jnp.float32),
        ] + [pltpu.SemaphoreType.DMA] * (2 * _NBUF),
        compiler_params=pltpu.CompilerParams(use_tc_tiling_on_sc=False,
                                             needs_layout_passes=False),
    )


def _sc_segment(*args):
    return _make_sc_segment()(*args)


def _sc_segment_body(hr, src_h, dst_h, w_h, zeros_h, out,
                     src_v, dst_v, w_v, rows_v, srows_v, acc, *sems):
    gsem = sems[:_NBUF]
    ssem = sems[_NBUF:]
    cid = lax.axis_index("c")
    sid = lax.axis_index("s")
    wid = sid * _NC + cid

    c1 = pltpu.async_copy(src_h.at[wid], src_v, gsem[0])
    c2 = pltpu.async_copy(dst_h.at[wid], dst_v, gsem[1])
    c3 = pltpu.async_copy(w_h.at[wid], w_v, gsem[2])
    c4 = pltpu.async_copy(zeros_h, acc.at[pl.ds(sid * _RPT, _RPT)], gsem[3])
    c1.wait(); c2.wait(); c3.wait(); c4.wait()
    plsc.subcore_barrier()

    for b in range(_NBUF):
        pltpu.async_copy(hr.at[src_v.at[b]], rows_v.at[b], gsem[b])

    def scale(j, b):
        shift16 = jnp.full((16,), 16, jnp.int32)
        himask = jnp.full((16,), -65536, jnp.int32)

        def group_body(g, c):
            w16 = w_v[j, pl.ds(g * 16, 16)]
            for e16 in range(16):
                e = g * 16 + e16
                wsplat = _splat_lane(w16, e16)
                for k in range(_H // 32):
                    v = rows_v[b, e, pl.ds(k * 16, 16)]
                    lo = plsc.bitcast(lax.shift_left(v, shift16), jnp.float32)
                    hi = plsc.bitcast(v & himask, jnp.float32)
                    srows_v[b, e, pl.ds(k * 32, 16)] = lo * wsplat
                    srows_v[b, e, pl.ds(k * 32 + 16, 16)] = hi * wsplat
            return c

        lax.fori_loop(0, _CH // 16, group_body, 0)

    def outer_body(go, carry):
        for b in range(_NBUF):
            j = go * _NBUF + b
            pltpu.make_async_copy(
                hr.at[pl.ds(0, _CH)], rows_v.at[b], gsem[b]).wait()
            @pl.when(j >= _NBUF)
            def _drain():
                pltpu.make_async_copy(
                    srows_v.at[b], acc.at[dst_v.at[0]], ssem[b]).wait()
            scale(j, b)
            pltpu.async_copy(srows_v.at[b], acc.at[dst_v.at[j]], ssem[b],
                             add=True)
            jn = j + _NBUF

            @pl.when(jn < _NCHUNK)
            def _refill():
                pltpu.async_copy(hr.at[src_v.at[jn]], rows_v.at[b], gsem[b])
        return carry

    lax.fori_loop(0, _NOUT, outer_body, 0)
    for b in range(_NBUF):
        pltpu.make_async_copy(
            srows_v.at[b], acc.at[dst_v.at[0]], ssem[b]).wait()
    plsc.subcore_barrier()

    pltpu.sync_copy(acc.at[pl.ds(sid * _RPT, _RPT)], out.at[cid, sid])


def _dot_t(a, w):
    return lax.dot_general(a, w, (((1,), (1,)), ((), ())),
                           preferred_element_type=jnp.float32)


def _pre_body(x_ref, wr_ref, ws_ref, hr_ref, hs_ref):
    x = x_ref[...]
    hr_ref[...] = _dot_t(x, wr_ref[...]).astype(jnp.bfloat16)
    hs_ref[...] = _dot_t(x, ws_ref[...])


def _mid_body(p_ref, hs_ref, b_ref, wr_ref, ws_ref, hr_ref, hs2_ref):
    h = jnp.maximum(p_ref[0, :_N] + p_ref[1, :_N] + hs_ref[...] + b_ref[...],
                    0.0)
    hr_ref[...] = _dot_t(h, wr_ref[...]).astype(jnp.bfloat16)
    hs2_ref[...] = _dot_t(h, ws_ref[...])


def _relu_body(p_ref, hs_ref, b_ref, pm_ref, h_ref, hp_ref):
    h = jnp.maximum(p_ref[0, :_N] + p_ref[1, :_N] + hs_ref[...]
                    + b_ref[...], 0.0)
    h_ref[...] = h
    hp_ref[...] = _dot_t(h, pm_ref[...]).astype(jnp.bfloat16)


def _pack_i32(hb):
    return lax.bitcast_convert_type(hb.reshape(_N, _H // 2, 2), jnp.int32)


def _final_body(p_ref, h_ref, b_ref, wr_ref, ws_ref, out_ref):
    agg = p_ref[0, :_N] + p_ref[1, :_N]
    out_ref[...] = (_dot_t(agg, wr_ref[...]) + b_ref[...]
                    + _dot_t(h_ref[...], ws_ref[...]))


def _f32(*shape):
    return jax.ShapeDtypeStruct(shape, jnp.float32)


def kernel(x, edge_index, edge_attr, W1r, b1, W1s, W2r, b2, W2s, W3r, b3, W3s):
    perm = jnp.asarray(_PERM, jnp.int32)
    pmat = jnp.eye(_H, dtype=jnp.float32)[perm]
    src = edge_index[0].reshape(_NW, _NCHUNK, _CH)
    dst = edge_index[1].reshape(_NW, _NCHUNK, _CH)
    w = edge_attr.reshape(_NW, _NCHUNK, _CH)
    zeros = jnp.zeros((_RPT, _H), jnp.float32)

    hr1, hs1 = pl.pallas_call(
        _pre_body,
        out_shape=[jax.ShapeDtypeStruct((_N, _H), jnp.bfloat16),
                   _f32(_N, _H)])(x, W1r[perm], W1s)
    p1 = _sc_segment(_pack_i32(hr1), src, dst, w,
                     zeros).reshape(_NC, _NPAD, _H)

    hr2, hs2 = pl.pallas_call(
        _mid_body,
        out_shape=[jax.ShapeDtypeStruct((_N, _H), jnp.bfloat16),
                   _f32(_N, _H)])(
            p1, hs1, b1.reshape(1, _H), W2r[perm], W2s)
    p2 = _sc_segment(_pack_i32(hr2), src, dst, w,
                     zeros).reshape(_NC, _NPAD, _H)

    h2, h2p = pl.pallas_call(
        _relu_body,
        out_shape=[_f32(_N, _H),
                   jax.ShapeDtypeStruct((_N, _H), jnp.bfloat16)])(
            p2, hs2, b2.reshape(1, _H), pmat)
    p3 = _sc_segment(_pack_i32(h2p), src, dst, w,
                     zeros).reshape(_NC, _NPAD, _H)

    out = pl.pallas_call(
        _final_body, out_shape=_f32(_N, _D))(
            p3, h2, b3.reshape(1, _D), W3r, W3s)
    return out


# speedup vs baseline: 1.7976x; 1.7976x over previous
import functools

import jax
import jax.numpy as jnp
from jax import lax
from jax.experimental import pallas as pl
from jax.experimental.pallas import tpu as pltpu
from jax.experimental.pallas import tpu_sc as plsc

_N = 10000
_E = 320000
_D = 128
_H = 64

_NC = 2
_NS = 16
_NW = _NC * _NS
_EPT = _E // _NW
_CH = 80
_NCHUNK = _EPT // _CH
_NPAD = 10240
_RPT = _NPAD // _NS

_GATHER_DNUMS = lax.GatherDimensionNumbers(
    offset_dims=(), collapsed_slice_dims=(0,), start_index_map=(0,))


def _splat_lane(vec, lane):
    idx = jnp.full((16, 1), lane, jnp.int32)
    return lax.gather(vec, idx, _GATHER_DNUMS, (1,),
                      mode=lax.GatherScatterMode.PROMISE_IN_BOUNDS)


_NBUF = 5
_NOUT = _NCHUNK // _NBUF


@functools.cache
def _make_sc_segment():
    mesh = plsc.VectorSubcoreMesh(core_axis_name="c", subcore_axis_name="s")
    return pl.kernel(
        _sc_segment_body,
        out_type=jax.ShapeDtypeStruct((_NC, _NS, _RPT, _H), jnp.float32),
        mesh=mesh,
        scratch_types=[
            pltpu.VMEM((_NCHUNK, _CH), jnp.int32),
            pltpu.VMEM((_NCHUNK, _CH), jnp.int32),
            pltpu.VMEM((_NCHUNK, _CH), jnp.float32),
            pltpu.VMEM((_NBUF, _CH, _H), jnp.float32),
            pltpu.VMEM((_NBUF, _CH, _H), jnp.float32),
            pltpu.VMEM_SHARED((_NPAD, _H), jnp.float32),
        ] + [pltpu.SemaphoreType.DMA] * (2 * _NBUF),
        compiler_params=pltpu.CompilerParams(use_tc_tiling_on_sc=False),
    )


def _sc_segment(*args):
    return _make_sc_segment()(*args)


def _sc_segment_body(hr, src_h, dst_h, w_h, zeros_h, out,
                     src_v, dst_v, w_v, rows_v, srows_v, acc, *sems):
    gsem = sems[:_NBUF]
    ssem = sems[_NBUF:]
    cid = lax.axis_index("c")
    sid = lax.axis_index("s")
    wid = sid * _NC + cid

    c1 = pltpu.async_copy(src_h.at[wid], src_v, gsem[0])
    c2 = pltpu.async_copy(dst_h.at[wid], dst_v, gsem[1])
    c3 = pltpu.async_copy(w_h.at[wid], w_v, gsem[2])
    c4 = pltpu.async_copy(zeros_h, acc.at[pl.ds(sid * _RPT, _RPT)], gsem[3])
    c1.wait(); c2.wait(); c3.wait(); c4.wait()
    plsc.subcore_barrier()

    for b in range(_NBUF):
        pltpu.async_copy(hr.at[src_v.at[b]], rows_v.at[b], gsem[b])

    def scale(j, b):
        def group_body(g, c):
            w16 = w_v[j, pl.ds(g * 16, 16)]
            for e16 in range(16):
                e = g * 16 + e16
                wsplat = _splat_lane(w16, e16)
                for f in range(_H // 16):
                    srows_v[b, e, pl.ds(f * 16, 16)] = (
                        rows_v[b, e, pl.ds(f * 16, 16)] * wsplat)
            return c

        lax.fori_loop(0, _CH // 16, group_body, 0)

    def outer_body(go, carry):
        for b in range(_NBUF):
            j = go * _NBUF + b
            pltpu.make_async_copy(
                hr.at[pl.ds(0, _CH)], rows_v.at[b], gsem[b]).wait()
            @pl.when(j >= _NBUF)
            def _drain():
                pltpu.make_async_copy(
                    srows_v.at[b], acc.at[dst_v.at[0]], ssem[b]).wait()
            scale(j, b)
            pltpu.async_copy(srows_v.at[b], acc.at[dst_v.at[j]], ssem[b],
                             add=True)
            jn = j + _NBUF

            @pl.when(jn < _NCHUNK)
            def _refill():
                pltpu.async_copy(hr.at[src_v.at[jn]], rows_v.at[b], gsem[b])
        return carry

    lax.fori_loop(0, _NOUT, outer_body, 0)
    for b in range(_NBUF):
        pltpu.make_async_copy(
            srows_v.at[b], acc.at[dst_v.at[0]], ssem[b]).wait()
    plsc.subcore_barrier()

    pltpu.sync_copy(acc.at[pl.ds(sid * _RPT, _RPT)], out.at[cid, sid])


def _dot_t(a, w):
    return lax.dot_general(a, w, (((1,), (1,)), ((), ())),
                           preferred_element_type=jnp.float32)


def _pre_body(x_ref, wr_ref, ws_ref, hr_ref, hs_ref):
    x = x_ref[...]
    hr_ref[...] = _dot_t(x, wr_ref[...])
    hs_ref[...] = _dot_t(x, ws_ref[...])


def _mid_body(p_ref, hs_ref, b_ref, wr_ref, ws_ref, hr_ref, hs2_ref):
    h = jnp.maximum(p_ref[0, :_N] + p_ref[1, :_N] + hs_ref[...] + b_ref[...],
                    0.0)
    hr_ref[...] = _dot_t(h, wr_ref[...])
    hs2_ref[...] = _dot_t(h, ws_ref[...])


def _relu_body(p_ref, hs_ref, b_ref, h_ref):
    h_ref[...] = jnp.maximum(p_ref[0, :_N] + p_ref[1, :_N] + hs_ref[...]
                             + b_ref[...], 0.0)


def _final_body(p_ref, h_ref, b_ref, wr_ref, ws_ref, out_ref):
    agg = p_ref[0, :_N] + p_ref[1, :_N]
    out_ref[...] = (_dot_t(agg, wr_ref[...]) + b_ref[...]
                    + _dot_t(h_ref[...], ws_ref[...]))


def _f32(*shape):
    return jax.ShapeDtypeStruct(shape, jnp.float32)


def kernel(x, edge_index, edge_attr, W1r, b1, W1s, W2r, b2, W2s, W3r, b3, W3s):
    src = edge_index[0].reshape(_NW, _NCHUNK, _CH)
    dst = edge_index[1].reshape(_NW, _NCHUNK, _CH)
    w = edge_attr.reshape(_NW, _NCHUNK, _CH)
    zeros = jnp.zeros((_RPT, _H), jnp.float32)

    hr1, hs1 = pl.pallas_call(
        _pre_body, out_shape=[_f32(_N, _H), _f32(_N, _H)])(x, W1r, W1s)
    p1 = _sc_segment(hr1, src, dst, w, zeros).reshape(_NC, _NPAD, _H)

    hr2, hs2 = pl.pallas_call(
        _mid_body, out_shape=[_f32(_N, _H), _f32(_N, _H)])(
            p1, hs1, b1.reshape(1, _H), W2r, W2s)
    p2 = _sc_segment(hr2, src, dst, w, zeros).reshape(_NC, _NPAD, _H)

    h2 = pl.pallas_call(
        _relu_body, out_shape=_f32(_N, _H))(p2, hs2, b2.reshape(1, _H))
    p3 = _sc_segment(h2, src, dst, w, zeros).reshape(_NC, _NPAD, _H)

    out = pl.pallas_call(
        _final_body, out_shape=_f32(_N, _D))(
            p3, h2, b3.reshape(1, _D), W3r, W3s)
    return out
